# baseline (speedup 1.0000x reference)
import jax
import jax.numpy as jnp
from jax.experimental import pallas as pl
from jax.experimental.pallas import tpu as pltpu

M_CAP = 1024
DIM = 512
BATCH = 64
INIT_SIZE = M_CAP - BATCH
A_T = 0.35
H_T = 0.3


def _gwr_body(scal_ref, data_ref, V_in, h_in, E_in, V_out, h_out, acts_out,
              E_scr, d8_scr):
    V_out[...] = V_in[...]
    h_out[...] = h_in[...]
    E_scr[...] = E_in[...]

    eps_b = scal_ref[0]
    eps_n = scal_ref[1]
    tau_b = scal_ref[2]
    tau_n = scal_ref[3]
    kappa = scal_ref[4]

    i8 = (jax.lax.broadcasted_iota(jnp.int32, (8, 128), 0) * 128
          + jax.lax.broadcasted_iota(jnp.int32, (8, 128), 1))
    irow = jax.lax.broadcasted_iota(jnp.int32, (1, M_CAP), 1)
    iacts = jax.lax.broadcasted_iota(jnp.int32, (1, BATCH), 1)

    def dist8(xv, Vv):
        d2 = jnp.sum((xv - Vv) * (xv - Vv), axis=1, keepdims=True)
        return jnp.reshape(jnp.reshape(d2, (1, M_CAP)), (8, 128))

    x0 = data_ref[pl.ds(0, 1), :]
    d8_0 = dist8(x0, V_out[...])

    def step(t, carry):
        size, acts, d8 = carry
        x = data_ref[pl.ds(t, 1), :]
        tn = jnp.minimum(t + 1, BATCH - 1)
        xn = data_ref[pl.ds(tn, 1), :]
        d = jnp.sqrt(d8 + 1e-12)
        d = jnp.where(i8 < size, d, jnp.inf)
        h8 = h_out[...]
        m = jnp.min(d)
        eqm = d == m
        b = jnp.min(jnp.where(eqm, i8, M_CAP))
        cnt = jnp.sum(eqm.astype(jnp.int32))
        m2x = jnp.min(jnp.where(eqm, jnp.inf, d))
        im = jnp.where(eqm, i8, M_CAP)
        s_tie = jnp.min(jnp.where(im == b, M_CAP, im))
        s_m2 = jnp.min(jnp.where(d == m2x, i8, M_CAP))
        hb = jnp.sum(jnp.where(i8 == b, h8, 0.0))
        s = jnp.where(cnt >= 2, s_tie, s_m2)
        a = jnp.exp(-m)
        insert = (a < A_T) & (hb < H_T) & (size < M_CAP)

        @pl.when(insert)
        def _():
            r = size
            Vb = V_out[pl.ds(b, 1), :]
            V_out[pl.ds(r, 1), :] = (x + Vb) * 0.5
            h_out[...] = jnp.where(i8 == r, 1.0, h8)
            erb = E_scr[pl.ds(b, 1), :]
            E_scr[pl.ds(b, 1), :] = jnp.where(
                irow == s, -1, jnp.where(irow == r, 0, erb))
            ers = E_scr[pl.ds(s, 1), :]
            E_scr[pl.ds(s, 1), :] = jnp.where(
                irow == b, -1, jnp.where(irow == r, 0, ers))
            E_scr[pl.ds(r, 1), :] = jnp.where(
                (irow == b) | (irow == s), 0, -1).astype(jnp.int32)
            d8_scr[...] = dist8(xn, V_out[...])

        @pl.when(jnp.logical_not(insert))
        def _():
            er2 = jnp.where(irow == s, 0, E_scr[pl.ds(b, 1), :])
            E_scr[pl.ds(b, 1), :] = er2
            ers = E_scr[pl.ds(s, 1), :]
            E_scr[pl.ds(s, 1), :] = jnp.where(irow == b, 0, ers)
            nb_r = (er2 > -1) & (irow != b)
            nbf_row = nb_r.astype(jnp.float32)
            nb8 = jnp.reshape(nbf_row, (8, 128))
            c_row = nbf_row * (eps_n * jnp.reshape(h8, (1, M_CAP)))
            c_col = jnp.reshape(c_row, (M_CAP, 1))

            hb_new = hb + tau_b * kappa * (1.0 - hb) - tau_b
            Vb = V_out[pl.ds(b, 1), :]
            Vb_new = Vb + eps_b * hb * (x - Vb)
            dbn = xn - Vb_new
            d2b = jnp.sum(dbn * dbn)
            hn8 = h8 + nb8 * (tau_n * kappa * (1.0 - h8) - tau_n)
            h_out[...] = jnp.where(i8 == b, hb_new, hn8)
            Vv = V_out[...]
            Vnew = Vv + c_col * (x - Vv)
            V_out[...] = Vnew
            d8n = dist8(xn, Vnew)
            V_out[pl.ds(b, 1), :] = Vb_new
            d8_scr[...] = jnp.where(i8 == b, d2b, d8n)

        acts = jnp.where(iacts == t, a, acts)
        size = jnp.where(insert, size + jnp.int32(1), size)
        return size, acts, d8_scr[...]

    size0 = jnp.int32(INIT_SIZE)
    acts0 = jnp.zeros((1, BATCH), jnp.float32)
    _, acts_f, _ = jax.lax.fori_loop(0, BATCH, step, (size0, acts0, d8_0))
    acts_out[...] = acts_f


def kernel(it, data, V, h, E, eps_b, eps_n, tau_b, tau_n, kappa):
    scal = jnp.stack([eps_b, eps_n, tau_b, tau_n, kappa]).astype(jnp.float32)
    Vf, hf, acts = pl.pallas_call(
        _gwr_body,
        out_shape=[
            jax.ShapeDtypeStruct((M_CAP, DIM), jnp.float32),
            jax.ShapeDtypeStruct((8, 128), jnp.float32),
            jax.ShapeDtypeStruct((1, BATCH), jnp.float32),
        ],
        in_specs=[
            pl.BlockSpec(memory_space=pltpu.SMEM),
            pl.BlockSpec(memory_space=pltpu.VMEM),
            pl.BlockSpec(memory_space=pltpu.VMEM),
            pl.BlockSpec(memory_space=pltpu.VMEM),
            pl.BlockSpec(memory_space=pltpu.VMEM),
        ],
        out_specs=[
            pl.BlockSpec(memory_space=pltpu.VMEM),
            pl.BlockSpec(memory_space=pltpu.VMEM),
            pl.BlockSpec(memory_space=pltpu.VMEM),
        ],
        scratch_shapes=[
            pltpu.VMEM((M_CAP, M_CAP), jnp.int32),
            pltpu.VMEM((8, 128), jnp.float32),
        ],
    )(scal, data, V, h.reshape(8, 128), E)
    return Vf, hf.reshape(M_CAP), acts.reshape(BATCH)


# speedup vs baseline: 1.1707x; 1.1707x over previous
import jax
import jax.numpy as jnp
from jax.experimental import pallas as pl
from jax.experimental.pallas import tpu as pltpu

M_CAP = 1024
DIM = 512
BATCH = 64
INIT_SIZE = M_CAP - BATCH
A_T = 0.35
H_T = 0.3


def _gwr_body(scal_ref, data_ref, V_in, h_in, E_in, V_out, h_out, acts_out,
              E_scr, d8_scr):
    V_out[...] = V_in[...]
    h_out[...] = h_in[...]
    E_scr[...] = E_in[...]

    eps_b = scal_ref[0]
    eps_n = scal_ref[1]
    tau_b = scal_ref[2]
    tau_n = scal_ref[3]
    kappa = scal_ref[4]

    i8 = (jax.lax.broadcasted_iota(jnp.int32, (8, 128), 0) * 128
          + jax.lax.broadcasted_iota(jnp.int32, (8, 128), 1))
    if8 = i8.astype(jnp.float32)
    MF = jnp.float32(M_CAP)
    irow = jax.lax.broadcasted_iota(jnp.int32, (1, M_CAP), 1)
    iacts = jax.lax.broadcasted_iota(jnp.int32, (1, BATCH), 1)

    def dist8(xv, Vv):
        d2 = jnp.sum((xv - Vv) * (xv - Vv), axis=1, keepdims=True)
        return jnp.reshape(jnp.reshape(d2, (1, M_CAP)), (8, 128))

    x0 = data_ref[pl.ds(0, 1), :]
    d8_0 = dist8(x0, V_out[...])

    def step(t, carry):
        size, acts, d8 = carry
        x = data_ref[pl.ds(t, 1), :]
        tn = jnp.minimum(t + 1, BATCH - 1)
        xn = data_ref[pl.ds(tn, 1), :]
        d = jnp.sqrt(d8 + 1e-12)
        d = jnp.where(i8 < size, d, jnp.inf)
        h8 = h_out[...]
        m = jnp.min(d)
        eqm = d == m
        b_f = jnp.min(jnp.where(eqm, if8, MF))
        cnt = jnp.sum(eqm.astype(jnp.float32))
        m2x = jnp.min(jnp.where(eqm, jnp.inf, d))
        im = jnp.where(eqm, if8, MF)
        s_tie = jnp.min(jnp.where(im == b_f, MF, im))
        s_m2 = jnp.min(jnp.where(d == m2x, if8, MF))
        hb = jnp.sum(jnp.where(if8 == b_f, h8, 0.0))
        s_f = jnp.where(cnt >= 2.0, s_tie, s_m2)
        b = b_f.astype(jnp.int32)
        s = s_f.astype(jnp.int32)
        a = jnp.exp(-m)
        insert = (a < A_T) & (hb < H_T) & (size < M_CAP)

        @pl.when(insert)
        def _():
            r = size
            Vb = V_out[pl.ds(b, 1), :]
            V_out[pl.ds(r, 1), :] = (x + Vb) * 0.5
            h_out[...] = jnp.where(i8 == r, 1.0, h8)
            erb = E_scr[pl.ds(b, 1), :]
            E_scr[pl.ds(b, 1), :] = jnp.where(
                irow == s, -1, jnp.where(irow == r, 0, erb))
            ers = E_scr[pl.ds(s, 1), :]
            E_scr[pl.ds(s, 1), :] = jnp.where(
                irow == b, -1, jnp.where(irow == r, 0, ers))
            E_scr[pl.ds(r, 1), :] = jnp.where(
                (irow == b) | (irow == s), 0, -1).astype(jnp.int32)
            d8_scr[...] = dist8(xn, V_out[...])

        @pl.when(jnp.logical_not(insert))
        def _():
            er2 = jnp.where(irow == s, 0, E_scr[pl.ds(b, 1), :])
            E_scr[pl.ds(b, 1), :] = er2
            ers = E_scr[pl.ds(s, 1), :]
            E_scr[pl.ds(s, 1), :] = jnp.where(irow == b, 0, ers)
            nb_r = (er2 > -1) & (irow != b)
            nbf_row = nb_r.astype(jnp.float32)
            nb8 = jnp.reshape(nbf_row, (8, 128))
            c_row = nbf_row * (eps_n * jnp.reshape(h8, (1, M_CAP)))
            c_col = jnp.reshape(c_row, (M_CAP, 1))

            hb_new = hb + tau_b * kappa * (1.0 - hb) - tau_b
            Vb = V_out[pl.ds(b, 1), :]
            Vb_new = Vb + eps_b * hb * (x - Vb)
            dbn = xn - Vb_new
            d2b = jnp.sum(dbn * dbn)
            hn8 = h8 + nb8 * (tau_n * kappa * (1.0 - h8) - tau_n)
            h_out[...] = jnp.where(i8 == b, hb_new, hn8)
            Vv = V_out[...]
            Vnew = Vv + c_col * (x - Vv)
            V_out[...] = Vnew
            d8n = dist8(xn, Vnew)
            V_out[pl.ds(b, 1), :] = Vb_new
            d8_scr[...] = jnp.where(i8 == b, d2b, d8n)

        acts = jnp.where(iacts == t, a, acts)
        size = jnp.where(insert, size + jnp.int32(1), size)
        return size, acts, d8_scr[...]

    size0 = jnp.int32(INIT_SIZE)
    acts0 = jnp.zeros((1, BATCH), jnp.float32)
    _, acts_f, _ = jax.lax.fori_loop(0, BATCH, step, (size0, acts0, d8_0))
    acts_out[...] = acts_f


def kernel(it, data, V, h, E, eps_b, eps_n, tau_b, tau_n, kappa):
    scal = jnp.stack([eps_b, eps_n, tau_b, tau_n, kappa]).astype(jnp.float32)
    Vf, hf, acts = pl.pallas_call(
        _gwr_body,
        out_shape=[
            jax.ShapeDtypeStruct((M_CAP, DIM), jnp.float32),
            jax.ShapeDtypeStruct((8, 128), jnp.float32),
            jax.ShapeDtypeStruct((1, BATCH), jnp.float32),
        ],
        in_specs=[
            pl.BlockSpec(memory_space=pltpu.SMEM),
            pl.BlockSpec(memory_space=pltpu.VMEM),
            pl.BlockSpec(memory_space=pltpu.VMEM),
            pl.BlockSpec(memory_space=pltpu.VMEM),
            pl.BlockSpec(memory_space=pltpu.VMEM),
        ],
        out_specs=[
            pl.BlockSpec(memory_space=pltpu.VMEM),
            pl.BlockSpec(memory_space=pltpu.VMEM),
            pl.BlockSpec(memory_space=pltpu.VMEM),
        ],
        scratch_shapes=[
            pltpu.VMEM((M_CAP, M_CAP), jnp.int32),
            pltpu.VMEM((8, 128), jnp.float32),
        ],
    )(scal, data, V, h.reshape(8, 128), E)
    return Vf, hf.reshape(M_CAP), acts.reshape(BATCH)
